# Initial kernel scaffold; baseline (speedup 1.0000x reference)
import jax
import jax.numpy as jnp
from jax.experimental import pallas as pl


def kernel(x, edge_index, enc_mask_token, e1_W1, e1_b1, e1_W2, e1_b2, e2_W1, e2_b1, e2_W2, e2_b2, d1_W1, d1_b1, d1_W2, d1_b2, d2_W1, d2_b1, d2_W2, d2_b2):
    raise NotImplementedError("write your pallas kernel here")


# speedup vs baseline: 3.1979x; 3.1979x over previous
import functools

import numpy as np
import jax
import jax.numpy as jnp
from jax import lax
from jax.experimental import pallas as pl
from jax.experimental.pallas import tpu as pltpu
from jax.experimental.pallas import tpu_sc as plsc

N = 10000
D = 128
E = 320000
NUM_MASK = 3000
NUM_NOISE = 300
NUM_TOKEN = 2700

NC, NS = 2, 16
NW = NC * NS
ECHUNK = 128
EPT_STEPS = 79
EPT = EPT_STEPS * ECHUNK
E_PAD = EPT * NW
NPAD = 10112
ZROWS = NPAD // NS
GCHUNK = 64
GROWS = 320
GSTEPS = GROWS // GCHUNK
N_PAD0 = GROWS * NW
XROWS = 8

_const_cache = {}


def _mask_constants():
    if not _const_cache:
        key = jax.random.key(42)
        k1, k2, k3 = jax.random.split(key, 3)
        perm = np.asarray(jax.random.permutation(k1, N))
        mask_nodes = perm[:NUM_MASK]
        perm_mask = np.asarray(jax.random.permutation(k2, NUM_MASK))
        token_nodes = mask_nodes[perm_mask[:NUM_TOKEN]]
        noise_nodes = mask_nodes[perm_mask[NUM_MASK - NUM_NOISE:]]
        noise_chosen = np.asarray(jax.random.permutation(k3, N))[:NUM_NOISE]
        g = np.arange(N, dtype=np.int32)
        g[token_nodes] = N
        g[noise_nodes] = noise_chosen
        g_pad = np.zeros((N_PAD0,), np.int32)
        g_pad[:N] = g
        maskbit = np.zeros((N, 1), np.float32)
        maskbit[mask_nodes] = 1.0
        _const_cache["g_pad"] = g_pad
        _const_cache["maskbit"] = maskbit
    return _const_cache["g_pad"], _const_cache["maskbit"]


_mask_constants()


def _gather_rows_body(xcat, gidx, out, idx_v, rows_v, sem):
    wid = lax.axis_index("s") * NC + lax.axis_index("c")
    base = wid * GROWS

    def step(i, carry):
        off = base + i * GCHUNK
        pltpu.sync_copy(gidx.at[pl.ds(off, GCHUNK)], idx_v)
        pltpu.async_copy(xcat.at[idx_v], rows_v, sem).wait()
        pltpu.sync_copy(rows_v, out.at[pl.ds(off, GCHUNK)])
        return carry

    lax.fori_loop(0, GSTEPS, step, 0)


def _edge_agg_body(h, src, dst, zeros, out, sidx_v, didx_v, rows_v, acc, sem):
    c = lax.axis_index("c")
    s = lax.axis_index("s")
    wid = s * NC + c
    pltpu.sync_copy(zeros.at[pl.ds(s * ZROWS, ZROWS)],
                    acc.at[pl.ds(s * ZROWS, ZROWS)])
    plsc.subcore_barrier()
    base = wid * EPT

    def step(i, carry):
        off = base + i * ECHUNK
        pltpu.sync_copy(src.at[pl.ds(off, ECHUNK)], sidx_v)
        pltpu.sync_copy(dst.at[pl.ds(off, ECHUNK)], didx_v)
        pltpu.async_copy(h.at[sidx_v], rows_v, sem).wait()
        pltpu.sync_copy(rows_v, acc.at[didx_v], add=True)
        return carry

    lax.fori_loop(0, EPT_STEPS, step, 0)
    plsc.subcore_barrier()
    pltpu.sync_copy(acc.at[pl.ds(s * ZROWS, ZROWS)],
                    out.at[c, pl.ds(s * ZROWS, ZROWS)])


@functools.lru_cache(maxsize=None)
def _sc_kernels():
    mesh = plsc.VectorSubcoreMesh(
        core_axis_name="c", subcore_axis_name="s",
        num_cores=NC, num_subcores=NS)
    gather = pl.kernel(
        _gather_rows_body,
        out_type=jax.ShapeDtypeStruct((N_PAD0, D), jnp.float32),
        mesh=mesh,
        scratch_types=[
            pltpu.VMEM((GCHUNK,), jnp.int32),
            pltpu.VMEM((GCHUNK, D), jnp.float32),
            pltpu.SemaphoreType.DMA,
        ],
    )
    agg = pl.kernel(
        _edge_agg_body,
        out_type=jax.ShapeDtypeStruct((NC, NPAD, D), jnp.float32),
        mesh=mesh,
        scratch_types=[
            pltpu.VMEM((ECHUNK,), jnp.int32),
            pltpu.VMEM((ECHUNK,), jnp.int32),
            pltpu.VMEM((ECHUNK, D), jnp.float32),
            pltpu.VMEM_SHARED((NPAD, D), jnp.float32),
            pltpu.SemaphoreType.DMA,
        ],
    )
    return gather, agg


NBLK = 10
BLK = N // NBLK


def _mlp_body(act, use_keep, *refs):
    if use_keep:
        (hin_ref, a0_ref, a1_ref, w1_ref, b1_ref, w2_ref, b2_ref,
         keep_ref, out_ref) = refs
    else:
        (hin_ref, a0_ref, a1_ref, w1_ref, b1_ref, w2_ref, b2_ref,
         out_ref) = refs
    z = hin_ref[...] + a0_ref[0] + a1_ref[0]
    hmid = jnp.maximum(
        jnp.dot(z, w1_ref[...], preferred_element_type=jnp.float32)
        + b1_ref[...], 0.0)
    o = jnp.dot(hmid, w2_ref[...], preferred_element_type=jnp.float32) \
        + b2_ref[...]
    if act:
        o = jnp.maximum(o, 0.0)
    if use_keep:
        o = o * keep_ref[...]
    out_ref[...] = o


def _row_spec():
    return pl.BlockSpec((BLK, D), lambda i: (i, 0))


def _agg_specs():
    return [pl.BlockSpec((1, BLK, D), lambda i: (0, i, 0)),
            pl.BlockSpec((1, BLK, D), lambda i: (1, i, 0))]


def _w_specs():
    return [pl.BlockSpec((D, D), lambda i: (0, 0)),
            pl.BlockSpec((1, D), lambda i: (0, 0)),
            pl.BlockSpec((D, D), lambda i: (0, 0)),
            pl.BlockSpec((1, D), lambda i: (0, 0))]


def _mlp(hin, agg, w1, b1, w2, b2, act, keep=None):
    use_keep = keep is not None
    in_specs = [_row_spec()] + _agg_specs() + _w_specs()
    args = [hin, agg, agg, w1, b1.reshape(1, D), w2, b2.reshape(1, D)]
    if use_keep:
        in_specs.append(pl.BlockSpec((BLK, 1), lambda i: (i, 0)))
        args.append(keep)
    return pl.pallas_call(
        functools.partial(_mlp_body, act, use_keep),
        grid=(NBLK,),
        in_specs=in_specs,
        out_specs=_row_spec(),
        out_shape=jax.ShapeDtypeStruct((N, D), jnp.float32),
    )(*args)


def _loss_body(hin_ref, a0_ref, a1_ref, w1_ref, b1_ref, w2_ref, b2_ref,
               x_ref, mb_ref, acc_ref):
    i = pl.program_id(0)
    z = hin_ref[...] + a0_ref[0] + a1_ref[0]
    hmid = jnp.maximum(
        jnp.dot(z, w1_ref[...], preferred_element_type=jnp.float32)
        + b1_ref[...], 0.0)
    o = jnp.dot(hmid, w2_ref[...], preferred_element_type=jnp.float32) \
        + b2_ref[...]
    xi = x_ref[...]
    s1 = jnp.sum(o * o, axis=1, keepdims=True)
    s2 = jnp.sum(xi * xi, axis=1, keepdims=True)
    dt = jnp.sum(o * xi, axis=1, keepdims=True)
    n1 = jnp.maximum(jnp.sqrt(s1), 1e-12)
    n2 = jnp.maximum(jnp.sqrt(s2), 1e-12)
    cerr = 1.0 - dt / (n1 * n2)
    part = jnp.sum(mb_ref[...] * cerr * cerr) * (1.0 / NUM_MASK)

    @pl.when(i == 0)
    def _init():
        acc_ref[...] = jnp.zeros_like(acc_ref)

    acc_ref[...] = acc_ref[...] + part


def _loss_mlp(hin, agg, w1, b1, w2, b2, x, maskbit):
    in_specs = ([_row_spec()] + _agg_specs() + _w_specs()
                + [_row_spec(), pl.BlockSpec((BLK, 1), lambda i: (i, 0))])
    out = pl.pallas_call(
        _loss_body,
        grid=(NBLK,),
        in_specs=in_specs,
        out_specs=pl.BlockSpec((1, 1), lambda i: (0, 0)),
        out_shape=jax.ShapeDtypeStruct((1, 1), jnp.float32),
    )(hin, agg, agg, w1, b1.reshape(1, D), w2, b2.reshape(1, D), x, maskbit)
    return out[0, 0]


def kernel(x, edge_index, enc_mask_token,
           e1_W1, e1_b1, e1_W2, e1_b2,
           e2_W1, e2_b1, e2_W2, e2_b2,
           d1_W1, d1_b1, d1_W2, d1_b2,
           d2_W1, d2_b1, d2_W2, d2_b2):
    g_pad_np, maskbit_np = _mask_constants()
    g_pad = jnp.asarray(g_pad_np)
    maskbit = jnp.asarray(maskbit_np)
    keep = 1.0 - maskbit

    src = edge_index[0]
    dst = edge_index[1]
    pad = E_PAD - E
    src_p = jnp.concatenate([src, jnp.zeros((pad,), jnp.int32)])
    dst_p = jnp.concatenate([dst, jnp.full((pad,), N, jnp.int32)])
    xcat = jnp.concatenate(
        [x, jnp.broadcast_to(enc_mask_token, (XROWS, D))], axis=0)
    zeros = jnp.zeros((NPAD, D), jnp.float32)

    _gather_rows, _edge_agg = _sc_kernels()
    out_x = _gather_rows(xcat, g_pad)[:N]
    a = _edge_agg(out_x, src_p, dst_p, zeros)
    h1 = _mlp(out_x, a, e1_W1, e1_b1, e1_W2, e1_b2, act=True)
    a = _edge_agg(h1, src_p, dst_p, zeros)
    rep = _mlp(h1, a, e2_W1, e2_b1, e2_W2, e2_b2, act=False, keep=keep)
    a = _edge_agg(rep, src_p, dst_p, zeros)
    h3 = _mlp(rep, a, d1_W1, d1_b1, d1_W2, d1_b2, act=True)
    a = _edge_agg(h3, src_p, dst_p, zeros)
    return _loss_mlp(h3, a, d2_W1, d2_b1, d2_W2, d2_b2, x, maskbit)
